# Initial kernel scaffold; baseline (speedup 1.0000x reference)
import jax
import jax.numpy as jnp
from jax.experimental import pallas as pl


def kernel(ego_embedding, edge_index, edge_weight, W, b):
    raise NotImplementedError("write your pallas kernel here")


# speedup vs baseline: 4.1611x; 4.1611x over previous
import functools

import jax
import jax.numpy as jnp
from jax import lax
from jax.experimental import pallas as pl
from jax.experimental.pallas import tpu as pltpu
from jax.experimental.pallas import tpu_sc as plsc

N_NODES = 10000
N_EDGES = 320000
DIM = 128
LANES = 16
VPR = DIM // LANES

NC, NS = 2, 16
NW = NC * NS
EDGES_PER_W = N_EDGES // NW
K = 80
NCHUNK = EDGES_PER_W // K
N_PAD = 10240
ROWS_PER_TILE = N_PAD // NS
CCH = 128
NCC = ROWS_PER_TILE // CCH


def _mm_body(x_ref, w_ref, b_ref, o_ref):
    o_ref[...] = lax.dot_general(
        x_ref[...], w_ref[...], (((1,), (1,)), ((), ())),
        preferred_element_type=jnp.float32) + b_ref[...]


def _linear(x, w, b2d):
    blk = 2000
    return pl.pallas_call(
        _mm_body,
        grid=(N_NODES // blk,),
        in_specs=[
            pl.BlockSpec((blk, DIM), lambda i: (i, 0)),
            pl.BlockSpec((DIM, DIM), lambda i: (0, 0)),
            pl.BlockSpec((1, DIM), lambda i: (0, 0)),
        ],
        out_specs=pl.BlockSpec((blk, DIM), lambda i: (i, 0)),
        out_shape=jax.ShapeDtypeStruct((N_NODES, DIM), jnp.float32),
    )(x, w, b2d)


def _add_body(p_ref, o_ref):
    o_ref[...] = p_ref[0] + p_ref[1]


def _combine(parts):
    blk = 2000
    return pl.pallas_call(
        _add_body,
        grid=(N_NODES // blk,),
        in_specs=[pl.BlockSpec((NC, blk, DIM), lambda i: (0, i, 0))],
        out_specs=pl.BlockSpec((blk, DIM), lambda i: (i, 0)),
        out_shape=jax.ShapeDtypeStruct((N_NODES, DIM), jnp.float32),
    )(parts)


def _sc_body(hidden, src, dst, w, part, src_v, dst_v, w_v, rows_v, zb_v, acc,
             sem):
    c = lax.axis_index("c")
    s = lax.axis_index("s")
    wid = s * NC + c

    z = jnp.zeros((LANES,), jnp.float32)

    @pl.loop(0, CCH)
    def _(i):
        for j in range(VPR):
            zb_v[i, pl.ds(j * LANES, LANES)] = z

    for t in range(NCC):
        pltpu.sync_copy(zb_v, acc.at[pl.ds(s * ROWS_PER_TILE + t * CCH, CCH)])
    plsc.subcore_barrier()

    ebase = wid * EDGES_PER_W

    @pl.loop(0, NCHUNK)
    def _(g):
        base = ebase + g * K
        pltpu.sync_copy(src.at[pl.ds(base, K)], src_v)
        pltpu.sync_copy(dst.at[pl.ds(base, K)], dst_v)
        pltpu.sync_copy(w.at[pl.ds(base, K)], w_v)
        pltpu.async_copy(hidden.at[src_v], rows_v, sem).wait()

        @pl.loop(0, K // LANES)
        def _(t):
            wv16 = w_v[pl.ds(t * LANES, LANES)]
            for l in range(LANES):
                wl = wv16[l]
                e = t * LANES + l
                for j in range(VPR):
                    sl = pl.ds(j * LANES, LANES)
                    rows_v[e, sl] = rows_v[e, sl] * wl

        pltpu.sync_copy(rows_v, acc.at[dst_v], add=True)

    plsc.subcore_barrier()
    for t in range(NCC):
        off = s * ROWS_PER_TILE + t * CCH
        pltpu.sync_copy(acc.at[pl.ds(off, CCH)], zb_v)
        pltpu.sync_copy(zb_v, part.at[c, pl.ds(off, CCH)])


_SC_OUT_TYPE = jax.ShapeDtypeStruct((NC, N_PAD, DIM), jnp.float32)
_SC_SCRATCH = [
    pltpu.VMEM((K,), jnp.int32),
    pltpu.VMEM((K,), jnp.int32),
    pltpu.VMEM((K,), jnp.float32),
    pltpu.VMEM((K, DIM), jnp.float32),
    pltpu.VMEM((CCH, DIM), jnp.float32),
    pltpu.VMEM_SHARED((N_PAD, DIM), jnp.float32),
    pltpu.SemaphoreType.DMA,
]

_sc_scatter = pl.kernel(
    _sc_body,
    out_type=_SC_OUT_TYPE,
    mesh=plsc.VectorSubcoreMesh(core_axis_name="c", subcore_axis_name="s"),
    scratch_types=_SC_SCRATCH,
)


def kernel(ego_embedding, edge_index, edge_weight, W, b):
    hidden = _linear(ego_embedding, W, b.reshape(1, DIM))
    parts = _sc_scatter(hidden, edge_index[0], edge_index[1], edge_weight)
    return _combine(parts)


# speedup vs baseline: 10.1404x; 2.4370x over previous
import jax
import jax.numpy as jnp
from jax import lax
from jax.experimental import pallas as pl
from jax.experimental.pallas import tpu as pltpu
from jax.experimental.pallas import tpu_sc as plsc

N_NODES = 10000
N_EDGES = 320000
DIM = 128
LANES = 16
VPR = DIM // LANES

NC, NS = 2, 16
NW = NC * NS
K = 80
EDGES_PER_W = N_EDGES // NW
NCHUNK = EDGES_PER_W // K
N_PAD = 10240
ROWS_PER_TILE = N_PAD // NS
CCH = 128
NCC = ROWS_PER_TILE // CCH
NIB = 4
STEADY = NIB + (NCHUNK - NIB - 2) // NIB * NIB


def _mm_body(x_ref, w_ref, b_ref, o_ref):
    o_ref[...] = lax.dot_general(
        x_ref[...], w_ref[...], (((1,), (1,)), ((), ())),
        preferred_element_type=jnp.float32) + b_ref[...]


def _linear(x, w, b2d):
    blk = 2000
    return pl.pallas_call(
        _mm_body,
        grid=(N_NODES // blk,),
        in_specs=[
            pl.BlockSpec((blk, DIM), lambda i: (i, 0)),
            pl.BlockSpec((DIM, DIM), lambda i: (0, 0)),
            pl.BlockSpec((1, DIM), lambda i: (0, 0)),
        ],
        out_specs=pl.BlockSpec((blk, DIM), lambda i: (i, 0)),
        out_shape=jax.ShapeDtypeStruct((N_NODES, DIM), jnp.float32),
    )(x, w, b2d)


def _add_body(p_ref, o_ref):
    o_ref[...] = p_ref[0] + p_ref[1]


def _combine(parts):
    blk = 2000
    return pl.pallas_call(
        _add_body,
        grid=(N_NODES // blk,),
        in_specs=[pl.BlockSpec((NC, blk, DIM), lambda i: (0, i, 0))],
        out_specs=pl.BlockSpec((blk, DIM), lambda i: (i, 0)),
        out_shape=jax.ShapeDtypeStruct((N_NODES, DIM), jnp.float32),
    )(parts)


def _sc_body(hidden, src, dst, w, part,
             src_v, dst_v, w_v, rows_v, zb_v, acc, sem_g, sem_s, sem_i):
    c = lax.axis_index("c")
    s = lax.axis_index("s")
    wid = s * NC + c
    ebase = wid * EDGES_PER_W

    def issue_idx(g, q):
        base = ebase + g * K
        pltpu.async_copy(src.at[pl.ds(base, K)], src_v.at[q], sem_i[q])
        pltpu.async_copy(dst.at[pl.ds(base, K)], dst_v.at[q], sem_i[q])
        pltpu.async_copy(w.at[pl.ds(base, K)], w_v.at[q], sem_i[q])

    def wait_idx(q):
        pltpu.make_async_copy(src.at[pl.ds(0, K)], src_v.at[q], sem_i[q]).wait()
        pltpu.make_async_copy(dst.at[pl.ds(0, K)], dst_v.at[q], sem_i[q]).wait()
        pltpu.make_async_copy(w.at[pl.ds(0, K)], w_v.at[q], sem_i[q]).wait()

    def issue_gather(q, b):
        pltpu.async_copy(hidden.at[src_v.at[q]], rows_v.at[b], sem_g[b])

    def wait_gather(q, b):
        pltpu.make_async_copy(hidden.at[src_v.at[q]], rows_v.at[b],
                              sem_g[b]).wait()

    def sync_scatter(q, b):
        pltpu.async_copy(rows_v.at[b], acc.at[dst_v.at[q]], sem_s[b],
                         add=True).wait()

    def multiply(q, b):
        @pl.loop(0, K // LANES)
        def _(t):
            wv16 = w_v[q, pl.ds(t * LANES, LANES)]
            for l in range(LANES):
                wl = wv16[l]
                e = t * LANES + l
                for j in range(VPR):
                    sl = pl.ds(j * LANES, LANES)
                    rows_v[b, e, sl] = rows_v[b, e, sl] * wl

    def chunk(g, qg, has_prev, has_next, has_next2):
        b = qg % 2
        if has_next:
            wait_idx((qg + 1) % NIB)
            issue_gather((qg + 1) % NIB, 1 - b)
        if has_next2:
            issue_idx(g + 2, (qg + 2) % NIB)
        wait_gather(qg % NIB, b)
        multiply(qg % NIB, b)
        sync_scatter(qg % NIB, b)

    issue_idx(0, 0)
    issue_idx(1, 1)
    wait_idx(0)
    issue_gather(0, 0)

    z = jnp.zeros((LANES,), jnp.float32)

    @pl.loop(0, CCH)
    def _(i):
        for j in range(VPR):
            zb_v[i, pl.ds(j * LANES, LANES)] = z

    for t in range(NCC):
        pltpu.sync_copy(zb_v, acc.at[pl.ds(s * ROWS_PER_TILE + t * CCH, CCH)])
    plsc.subcore_barrier()

    for g in range(NIB):
        chunk(g, g, g > 0, True, True)

    @pl.loop(NIB, STEADY, step=NIB)
    def _(g):
        for k in range(NIB):
            chunk(g + k, k, True, True, True)

    for g in range(STEADY, NCHUNK):
        chunk(g, g % NIB, True, g + 1 < NCHUNK, g + 2 < NCHUNK)

    plsc.subcore_barrier()
    for t in range(NCC):
        off = s * ROWS_PER_TILE + t * CCH
        pltpu.sync_copy(acc.at[pl.ds(off, CCH)], zb_v)
        pltpu.sync_copy(zb_v, part.at[c, pl.ds(off, CCH)])


_SC_OUT_TYPE = jax.ShapeDtypeStruct((NC, N_PAD, DIM), jnp.float32)
_SC_SCRATCH = [
    pltpu.VMEM((NIB, K), jnp.int32),
    pltpu.VMEM((NIB, K), jnp.int32),
    pltpu.VMEM((NIB, K), jnp.float32),
    pltpu.VMEM((2, K, DIM), jnp.float32),
    pltpu.VMEM((CCH, DIM), jnp.float32),
    pltpu.VMEM_SHARED((N_PAD, DIM), jnp.float32),
    [pltpu.SemaphoreType.DMA] * 2,
    [pltpu.SemaphoreType.DMA] * 2,
    [pltpu.SemaphoreType.DMA] * NIB,
]

_sc_scatter = pl.kernel(
    _sc_body,
    out_type=_SC_OUT_TYPE,
    mesh=plsc.VectorSubcoreMesh(core_axis_name="c", subcore_axis_name="s"),
    scratch_types=_SC_SCRATCH,
)


def kernel(ego_embedding, edge_index, edge_weight, W, b):
    hidden = _linear(ego_embedding, W, b.reshape(1, DIM))
    parts = _sc_scatter(hidden, edge_index[0], edge_index[1], edge_weight)
    return _combine(parts)
